# baseline (speedup 1.0000x reference)
import functools

import jax
import jax.numpy as jnp
from jax import lax
from jax.experimental import pallas as pl
from jax.experimental.pallas import tpu as pltpu
from jax.experimental.pallas import tpu_sc as plsc

NUM_EMBEDDINGS = 1000000
EMBED_DIM = 64
BATCH = 16384

_INFO = plsc.get_sparse_core_info()
_NC, _NS, _L = _INFO.num_cores, _INFO.num_subcores, _INFO.num_lanes
_NW = _NC * _NS
_BPW = BATCH // _NW
_CHUNK = 128
_NCHUNK = _BPW // _CHUNK


def _sc_kernel(nodes2d_hbm, table_hbm, out_hbm, idx_v, rows_v, outt_v, sem):
    wid = lax.axis_index("s") * _NC + lax.axis_index("c")
    base = wid * _BPW

    pltpu.sync_copy(nodes2d_hbm.at[pl.ds(wid * _NCHUNK, _NCHUNK)], idx_v)

    copies = []
    for q in range(_NCHUNK):
        copies.append(
            pltpu.async_copy(
                table_hbm.at[idx_v.at[q]],
                rows_v.at[pl.ds(q * _CHUNK, _CHUNK)],
                sem,
            )
        )
    for c in copies:
        c.wait()

    iota = lax.iota(jnp.int32, _L)

    def body(j, carry):
        for db in range(EMBED_DIM // _L):
            v = rows_v[j, pl.ds(db * _L, _L)]
            plsc.store_scatter(
                outt_v,
                [iota + (db * _L), jnp.full((_L,), j, dtype=jnp.int32)],
                v,
            )
        return carry

    lax.fori_loop(0, _BPW, body, 0, unroll=2)

    pltpu.sync_copy(outt_v, out_hbm.at[:, pl.ds(base, _BPW)])


@jax.jit
def _lookup_t(nodes, table):
    nodes2d = nodes.astype(jnp.int32).reshape(BATCH // _CHUNK, _CHUNK)
    mesh = plsc.VectorSubcoreMesh(core_axis_name="c", subcore_axis_name="s")
    f = functools.partial(
        pl.kernel,
        mesh=mesh,
        out_type=jax.ShapeDtypeStruct((EMBED_DIM, BATCH), jnp.float32),
        scratch_types=[
            pltpu.VMEM((_NCHUNK, _CHUNK), jnp.int32),
            pltpu.VMEM((_BPW, EMBED_DIM), jnp.float32),
            pltpu.VMEM((EMBED_DIM, _BPW), jnp.float32),
            pltpu.SemaphoreType.DMA,
        ],
        compiler_params=pltpu.CompilerParams(
            needs_layout_passes=False, use_tc_tiling_on_sc=False
        ),
    )(_sc_kernel)
    return f(nodes2d, table)


def kernel(nodes, table):
    return _lookup_t(nodes, table)


# speedup vs baseline: 1.0307x; 1.0307x over previous
import functools

import jax
import jax.numpy as jnp
from jax import lax
from jax.experimental import pallas as pl
from jax.experimental.pallas import tpu as pltpu
from jax.experimental.pallas import tpu_sc as plsc

NUM_EMBEDDINGS = 1000000
EMBED_DIM = 64
BATCH = 16384

_INFO = plsc.get_sparse_core_info()
_NC, _NS, _L = _INFO.num_cores, _INFO.num_subcores, _INFO.num_lanes
_NW = _NC * _NS
_BPW = BATCH // _NW
_CHUNK = 128
_NCHUNK = _BPW // _CHUNK
_PAD = _CHUNK + 1


def _sc_kernel(nodes2d_hbm, table_hbm, out_hbm, idx_v, rows_v, outt_v,
               sem_g, sem_o):
    wid = lax.axis_index("s") * _NC + lax.axis_index("c")
    base = wid * _BPW

    pltpu.sync_copy(nodes2d_hbm.at[pl.ds(wid * _NCHUNK, _NCHUNK)], idx_v)

    gathers = [
        pltpu.async_copy(table_hbm.at[idx_v.at[q]], rows_v.at[q], sem_g.at[q])
        for q in range(_NCHUNK)
    ]

    iota = lax.iota(jnp.int32, _L)
    col0 = jnp.zeros((_L,), jnp.int32)
    out_copies = []
    for q in range(_NCHUNK):
        gathers[q].wait()
        rows_q = rows_v.at[q]
        outt_q = outt_v.at[q]

        @plsc.parallel_loop(0, _CHUNK, carry=col0, unroll=4)
        def body(j, colv, rows_q=rows_q, outt_q=outt_q):
            for db in range(EMBED_DIM // _L):
                v = rows_q[j, pl.ds(db * _L, _L)]
                plsc.store_scatter(outt_q, [iota + (db * _L), colv], v)
            return colv + 1

        out_copies.append(
            pltpu.async_copy(
                outt_q.at[:, pl.ds(0, _CHUNK)],
                out_hbm.at[:, pl.ds(base + q * _CHUNK, _CHUNK)],
                sem_o,
            )
        )
    for c in out_copies:
        c.wait()


@jax.jit
def _lookup_t(nodes, table):
    nodes2d = nodes.astype(jnp.int32).reshape(BATCH // _CHUNK, _CHUNK)
    mesh = plsc.VectorSubcoreMesh(core_axis_name="c", subcore_axis_name="s")
    f = functools.partial(
        pl.kernel,
        mesh=mesh,
        out_type=jax.ShapeDtypeStruct((EMBED_DIM, BATCH), jnp.float32),
        scratch_types=[
            pltpu.VMEM((_NCHUNK, _CHUNK), jnp.int32),
            pltpu.VMEM((_NCHUNK, _CHUNK, EMBED_DIM), jnp.float32),
            pltpu.VMEM((_NCHUNK, EMBED_DIM, _PAD), jnp.float32),
            pltpu.SemaphoreType.DMA((_NCHUNK,)),
            pltpu.SemaphoreType.DMA,
        ],
        compiler_params=pltpu.CompilerParams(
            needs_layout_passes=False, use_tc_tiling_on_sc=False
        ),
    )(_sc_kernel)
    return f(nodes2d, table)


def kernel(nodes, table):
    return _lookup_t(nodes, table)
